# baseline (speedup 1.0000x reference)
import functools

import jax
import jax.numpy as jnp
from jax import lax
from jax.experimental import pallas as pl
from jax.experimental.pallas import tpu as pltpu
from jax.experimental.pallas import tpu_sc as plsc

NC = 2
NS = 16
NW = NC * NS
CH = 128


@functools.partial(jax.jit, static_argnames=("n_rows", "dim", "vocab"))
def _sc_gather(idx2d, table, n_rows, dim, vocab):
    b_per_w = n_rows // NW
    n_chunks = b_per_w // CH
    v_main = (vocab // (8 * NS)) * 8
    v_rem = vocab - v_main * NS

    def body(table_hbm, idx_hbm, out_hbm, shared_tab,
             ib0, ib1, ib2, rb0, rb1, rb2,
             gsem0, gsem1, gsem2, isem0, isem1, isem2, wsem0, wsem1, wsem2):
        cid = lax.axis_index("c")
        sid = lax.axis_index("s")
        wid = sid * NC + cid
        base = wid * b_per_w

        pltpu.sync_copy(
            table_hbm.at[pl.ds(sid * v_main, v_main)],
            shared_tab.at[pl.ds(sid * v_main, v_main)],
        )
        if v_rem:
            @pl.when(sid == 0)
            def _():
                pltpu.sync_copy(
                    table_hbm.at[pl.ds(NS * v_main, v_rem)],
                    shared_tab.at[pl.ds(NS * v_main, v_rem)],
                )
        plsc.subcore_barrier()

        rbs = (rb0, rb1, rb2)
        ibs = (ib0, ib1, ib2)
        gsems = (gsem0, gsem1, gsem2)
        isems = (isem0, isem1, isem2)
        wsems = (wsem0, wsem1, wsem2)
        chunk0 = wid * n_chunks
        n = n_chunks
        nl = n - (n % 3)

        def gather(c, slot):
            return pltpu.make_async_copy(
                shared_tab.at[ibs[slot].at[0]], rbs[slot], gsems[slot])

        def write(c, slot):
            return pltpu.make_async_copy(
                rbs[slot], out_hbm.at[pl.ds(base + c * CH, CH)], wsems[slot])

        def idx_load(c, slot):
            return pltpu.make_async_copy(
                idx_hbm.at[pl.ds(chunk0 + c, 1)], ibs[slot], isems[slot])

        for b in range(3):
            pltpu.sync_copy(idx_hbm.at[pl.ds(chunk0 + b, 1)], ibs[b])
        gather(0, 0).start()

        @pl.loop(0, nl, step=3)
        def _(j):
            for b in range(3):
                bn = (b + 1) % 3
                jj = j + b

                @pl.when(jj + 1 < n)
                def _():
                    @pl.when(jj >= 2)
                    def _():
                        write(jj - 2, bn).wait()
                        idx_load(jj + 1, bn).wait()
                    gather(jj + 1, bn).start()

                gather(jj, b).wait()
                write(jj, b).start()

                @pl.when(jj + 3 < n)
                def _():
                    idx_load(jj + 3, b).start()

        for t, c in enumerate(range(nl, n)):
            s = c % 3
            if t > 0:
                write(c - 3, s).wait()
                idx_load(c, s).wait()
                gather(c, s).start()
            gather(c, s).wait()
            write(c, s).start()
        for c in range(n - 3, n):
            write(c, c % 3).wait()

    mesh = plsc.VectorSubcoreMesh(core_axis_name="c", subcore_axis_name="s")
    f = pl.kernel(
        body,
        out_type=jax.ShapeDtypeStruct((n_rows, dim), jnp.float32),
        mesh=mesh,
        scratch_types=(
            [pltpu.VMEM_SHARED((vocab, dim), jnp.float32)]
            + [pltpu.VMEM((1, CH), jnp.int32)] * 3
            + [pltpu.VMEM((CH, dim), jnp.float32)] * 3
            + [pltpu.SemaphoreType.DMA] * 9
        ),
    )
    return f(table, idx2d)


def kernel(x, embedding):
    b, h = x.shape
    v, d = embedding.shape
    n_rows = b * h
    assert n_rows % (NW * CH * 2) == 0
    idx2d = x.reshape(n_rows // CH, CH)
    out = _sc_gather(idx2d, embedding, n_rows, d, v)
    return out.reshape(b, h, d)
